# baseline (speedup 1.0000x reference)
import jax
import jax.numpy as jnp
from jax import lax
from jax.experimental import pallas as pl
from jax.experimental.pallas import tpu as pltpu
from jax.experimental.pallas import tpu_sc as plsc

_N = 10000
_D = 128
_E = 320000
_OUT = 64
_NC = 2
_NS = 16
_NW = _NC * _NS
_C = 80
_EPT = _E // _NW
_NCH = _EPT // _C
_G = _C // 16
_PKH = 64
_RPT = _N // _NS
_HR = _N // 16
_TPAD = 1024
_TPT = _TPAD // _NS


def _sc_body(pki_h, feat_h, tid_h, twg_h, idn_h, zc_h,
             feat_o, cnt_o, deg_o,
             pki_v, sidxr, didxr, rows_a, rows_b, hist, idn_v, tid_v, twg_v,
             accf, accc, accd, sem_a, sem_b):
    cid = lax.axis_index("c")
    sid = lax.axis_index("s")
    wid = cid * _NS + sid
    r0 = sid * _RPT
    base = wid * _NCH
    zv = jnp.zeros((16,), jnp.float32)

    def zrow(i, carry):
        for g in range(_D // 16):
            rows_a[i, pl.ds(16 * g, 16)] = zv
        return carry

    lax.fori_loop(0, _C, zrow, 0)
    for k in range(_RPT // _C):
        pltpu.sync_copy(rows_a, accf.at[pl.ds(r0 + _C * k, _C)])
    pltpu.sync_copy(rows_a.at[pl.ds(0, _RPT % _C)],
                    accf.at[pl.ds(r0 + (_RPT // _C) * _C, _RPT % _C)])
    pltpu.sync_copy(zc_h.at[pl.ds(r0, _RPT)], accc.at[pl.ds(r0, _RPT)])

    @pl.when(sid == 0)
    def _():
        pltpu.sync_copy(zc_h.at[pl.ds(0, _HR)], accd)

    def zhist(i, carry):
        hist[i, pl.ds(0, 16)] = zv
        return carry

    lax.fori_loop(0, _HR, zhist, 0)
    pltpu.sync_copy(pki_h.at[pl.ds(base, _PKH)], pki_v)
    pltpu.sync_copy(idn_h, idn_v)
    pltpu.sync_copy(tid_h.at[pl.ds(sid * _TPT, _TPT)], tid_v.at[0])
    pltpu.sync_copy(twg_h.at[pl.ds(sid * _TPT, _TPT)], twg_v)
    plsc.subcore_barrier()

    def unpack(j, row, off):
        for g in range(_G):
            pk = pki_v[j - off, pl.ds(16 * g, 16)]
            dv = lax.shift_right_logical(pk, 16)
            sidxr[row, pl.ds(16 * g, 16)] = pk & 0xFFFF
            didxr[row, pl.ds(16 * g, 16)] = dv
            cnt, last = plsc.scan_count(dv)
            plsc.addupdate_scatter(
                hist, [lax.shift_right_logical(dv, 4), dv & 0xF],
                cnt.astype(jnp.float32), mask=last)

    unpack(0, 0, 0)
    unpack(1, 1, 0)
    pltpu.async_copy(feat_h.at[sidxr.at[0]], rows_a, sem_a)
    pltpu.async_copy(feat_h.at[sidxr.at[1]], rows_b, sem_b)

    def make_step(off):
        def step(jj, carry):
            j2 = 2 * jj + 2
            j3 = 2 * jj + 3
            pltpu.make_async_copy(feat_h.at[sidxr.at[0]], rows_a, sem_a).wait()
            pltpu.sync_copy(rows_a, accf.at[didxr.at[0]], add=True)
            unpack(j2, 0, off)
            pltpu.async_copy(feat_h.at[sidxr.at[0]], rows_a, sem_a)
            pltpu.make_async_copy(feat_h.at[sidxr.at[1]], rows_b, sem_b).wait()
            pltpu.sync_copy(rows_b, accf.at[didxr.at[1]], add=True)

            @pl.when(j3 < _NCH)
            def _():
                unpack(j3, 1, off)
                pltpu.async_copy(feat_h.at[sidxr.at[1]], rows_b, sem_b)

            return carry

        return step

    lax.fori_loop(0, 31, make_step(0), 0)
    pltpu.sync_copy(pki_h.at[pl.ds(base + _PKH, _NCH - _PKH)],
                    pki_v.at[pl.ds(0, _NCH - _PKH)])
    lax.fori_loop(31, _NCH // 2, make_step(_PKH), 0)
    pltpu.make_async_copy(feat_h.at[sidxr.at[0]], rows_a, sem_a).wait()
    pltpu.sync_copy(rows_a, accf.at[didxr.at[0]], add=True)
    for k in range(_HR // 125):
        pltpu.sync_copy(hist.at[pl.ds(125 * k, 125)], accd.at[idn_v.at[k]],
                        add=True)
    pltpu.sync_copy(twg_v, accc.at[tid_v.at[0]], add=True)
    plsc.subcore_barrier()
    o0 = cid * _N + r0
    pltpu.sync_copy(accf.at[pl.ds(r0, _RPT)], feat_o.at[pl.ds(o0, _RPT)])
    pltpu.sync_copy(accc.at[pl.ds(r0, _RPT)], cnt_o.at[pl.ds(o0, _RPT)])

    @pl.when(sid == 0)
    def _():
        pltpu.sync_copy(accd, deg_o.at[pl.ds(cid * _HR, _HR)])


def _tc_body(fp, cp, d0, d1, wp, pr, we, wr, out):
    f = fp[...]
    c = cp[...]
    agg = f[:_N] + f[_N:]
    deg = jnp.maximum(d0[...] + d1[...], 1.0)
    nrm = agg / deg
    h = jnp.maximum(jnp.dot(nrm, wp[...], preferred_element_type=jnp.float32), 0.0)
    wv = c[:_N, 1:2] + c[_N:, 1:2]
    tpv = jnp.sum(h * wv, axis=0, keepdims=True) / jnp.sum(wv)
    diff = pr[...] - tpv
    d2 = jnp.sum(diff * diff, axis=1, keepdims=True)
    oh = (d2 == jnp.min(d2)).astype(jnp.float32)
    wsel = jnp.sum(we[...] * oh[:, :, None], axis=0)
    x = jnp.maximum(jnp.dot(nrm, wsel, preferred_element_type=jnp.float32), 0.0)
    out[...] = jnp.dot(x, wr[...], preferred_element_type=jnp.float32)


def kernel(features, edge_index, test_ids, W_proj, expert_protos, W_expert, W_reg):
    pki = (edge_index[0] | (edge_index[1] << 16)).reshape(_NW * _NCH, _C)
    ntest = test_ids.shape[0]
    tid_p = jnp.concatenate(
        [test_ids.astype(jnp.int32), jnp.zeros((_TPAD - ntest,), jnp.int32)])
    twg = jnp.zeros((_TPAD, 16), jnp.float32).at[:ntest, 1].set(1.0)
    idn = jnp.arange(_HR, dtype=jnp.int32).reshape(_HR // 125, 125)
    zc = jnp.zeros((_N, 16), jnp.float32)

    sc_call = pl.kernel(
        _sc_body,
        out_type=[
            jax.ShapeDtypeStruct((_NC * _N, _D), jnp.float32),
            jax.ShapeDtypeStruct((_NC * _N, 16), jnp.float32),
            jax.ShapeDtypeStruct((_NC * _HR, 16), jnp.float32),
        ],
        mesh=plsc.VectorSubcoreMesh(core_axis_name="c", subcore_axis_name="s"),
        scratch_types=[
            pltpu.VMEM((_PKH, _C), jnp.int32),
            pltpu.VMEM((2, _C), jnp.int32),
            pltpu.VMEM((2, _C), jnp.int32),
            pltpu.VMEM((_C, _D), jnp.float32),
            pltpu.VMEM((_C, _D), jnp.float32),
            pltpu.VMEM((_HR, 16), jnp.float32),
            pltpu.VMEM((_HR // 125, 125), jnp.int32),
            pltpu.VMEM((1, _TPT), jnp.int32),
            pltpu.VMEM((_TPT, 16), jnp.float32),
            pltpu.VMEM_SHARED((_N, _D), jnp.float32),
            pltpu.VMEM_SHARED((_N, 16), jnp.float32),
            pltpu.VMEM_SHARED((_HR, 16), jnp.float32),
            pltpu.SemaphoreType.DMA,
            pltpu.SemaphoreType.DMA,
        ],
        compiler_params=pltpu.CompilerParams(use_tc_tiling_on_sc=False, needs_layout_passes=False),
    )
    feat_o, cnt_o, deg_o = sc_call(pki, features, tid_p, twg, idn, zc)

    d0 = deg_o[:_HR].reshape(_N, 1)
    d1 = deg_o[_HR:].reshape(_N, 1)
    out = pl.pallas_call(
        _tc_body,
        out_shape=jax.ShapeDtypeStruct((_N, _OUT), jnp.float32),
    )(feat_o, cnt_o, d0, d1, W_proj, expert_protos, W_expert, W_reg)
    return out


# speedup vs baseline: 1.0773x; 1.0773x over previous
import jax
import jax.numpy as jnp
from jax import lax
from jax.experimental import pallas as pl
from jax.experimental.pallas import tpu as pltpu
from jax.experimental.pallas import tpu_sc as plsc

_N = 10000
_D = 128
_E = 320000
_OUT = 64
_NC = 2
_NS = 16
_NW = _NC * _NS
_C = 80
_EPT = _E // _NW
_NCH = _EPT // _C
_G = _C // 16
_RPT = _N // _NS
_TPAD = 1024
_TPT = _TPAD // _NS


def _sc_body(pki_h, feat_h, tid_h, twg_h, ones_h, zc_h,
             feat_o, cnt_o,
             pki_v, sidxr, didxr, rows_a, rows_b, ones_v, tid_v, twg_v,
             accf, accc, sem_a, sem_b):
    cid = lax.axis_index("c")
    sid = lax.axis_index("s")
    wid = cid * _NS + sid
    r0 = sid * _RPT
    zv = jnp.zeros((16,), jnp.float32)

    def zrow(i, carry):
        for g in range(_D // 16):
            rows_a[i, pl.ds(16 * g, 16)] = zv
        return carry

    lax.fori_loop(0, _C, zrow, 0)
    for k in range(_RPT // _C):
        pltpu.sync_copy(rows_a, accf.at[pl.ds(r0 + _C * k, _C)])
    pltpu.sync_copy(rows_a.at[pl.ds(0, _RPT % _C)],
                    accf.at[pl.ds(r0 + (_RPT // _C) * _C, _RPT % _C)])
    pltpu.sync_copy(zc_h.at[pl.ds(r0, _RPT)], accc.at[pl.ds(r0, _RPT)])
    base = wid * _NCH
    pltpu.sync_copy(pki_h.at[pl.ds(base, _NCH)], pki_v)
    pltpu.sync_copy(ones_h, ones_v)
    pltpu.sync_copy(tid_h.at[pl.ds(sid * _TPT, _TPT)], tid_v.at[0])
    pltpu.sync_copy(twg_h.at[pl.ds(sid * _TPT, _TPT)], twg_v)
    plsc.subcore_barrier()

    def unpack(j, row):
        for g in range(_G):
            pk = pki_v[j, pl.ds(16 * g, 16)]
            sidxr[row, pl.ds(16 * g, 16)] = pk & 0xFFFF
            didxr[row, pl.ds(16 * g, 16)] = lax.shift_right_logical(pk, 16)

    unpack(0, 0)
    unpack(1, 1)
    pltpu.async_copy(feat_h.at[sidxr.at[0]], rows_a, sem_a)
    pltpu.async_copy(feat_h.at[sidxr.at[1]], rows_b, sem_b)

    def step(jj, carry):
        j2 = 2 * jj + 2
        j3 = 2 * jj + 3
        pltpu.make_async_copy(feat_h.at[sidxr.at[0]], rows_a, sem_a).wait()
        pltpu.sync_copy(rows_a, accf.at[didxr.at[0]], add=True)
        pltpu.sync_copy(ones_v, accc.at[didxr.at[0]], add=True)
        unpack(j2, 0)
        pltpu.async_copy(feat_h.at[sidxr.at[0]], rows_a, sem_a)
        pltpu.make_async_copy(feat_h.at[sidxr.at[1]], rows_b, sem_b).wait()
        pltpu.sync_copy(rows_b, accf.at[didxr.at[1]], add=True)
        pltpu.sync_copy(ones_v, accc.at[didxr.at[1]], add=True)

        @pl.when(j3 < _NCH)
        def _():
            unpack(j3, 1)
            pltpu.async_copy(feat_h.at[sidxr.at[1]], rows_b, sem_b)

        return carry

    lax.fori_loop(0, _NCH // 2, step, 0)
    pltpu.make_async_copy(feat_h.at[sidxr.at[0]], rows_a, sem_a).wait()
    pltpu.sync_copy(rows_a, accf.at[didxr.at[0]], add=True)
    pltpu.sync_copy(ones_v, accc.at[didxr.at[0]], add=True)
    pltpu.sync_copy(twg_v, accc.at[tid_v.at[0]], add=True)
    plsc.subcore_barrier()
    o0 = cid * _N + r0
    pltpu.sync_copy(accf.at[pl.ds(r0, _RPT)], feat_o.at[pl.ds(o0, _RPT)])
    pltpu.sync_copy(accc.at[pl.ds(r0, _RPT)], cnt_o.at[pl.ds(o0, _RPT)])


def _tc_body(fp, cp, wp, pr, we, wr, out):
    f = fp[...]
    c = cp[...]
    agg = f[:_N] + f[_N:]
    deg = jnp.maximum(c[:_N, 0:1] + c[_N:, 0:1], 1.0)
    nrm = agg / deg
    h = jnp.maximum(jnp.dot(nrm, wp[...], preferred_element_type=jnp.float32), 0.0)
    wv = c[:_N, 1:2] + c[_N:, 1:2]
    tpv = jnp.sum(h * wv, axis=0, keepdims=True) / jnp.sum(wv)
    diff = pr[...] - tpv
    d2 = jnp.sum(diff * diff, axis=1, keepdims=True)
    oh = (d2 == jnp.min(d2)).astype(jnp.float32)
    wsel = jnp.sum(we[...] * oh[:, :, None], axis=0)
    x = jnp.maximum(jnp.dot(nrm, wsel, preferred_element_type=jnp.float32), 0.0)
    out[...] = jnp.dot(x, wr[...], preferred_element_type=jnp.float32)


def kernel(features, edge_index, test_ids, W_proj, expert_protos, W_expert, W_reg):
    pki = (edge_index[0] | (edge_index[1] << 16)).reshape(_NW * _NCH, _C)
    ntest = test_ids.shape[0]
    tid_p = jnp.concatenate(
        [test_ids.astype(jnp.int32), jnp.zeros((_TPAD - ntest,), jnp.int32)])
    twg = jnp.zeros((_TPAD, 16), jnp.float32).at[:ntest, 1].set(1.0)
    ones_c = jnp.zeros((_C, 16), jnp.float32).at[:, 0].set(1.0)
    zc = jnp.zeros((_N, 16), jnp.float32)

    sc_call = pl.kernel(
        _sc_body,
        out_type=[
            jax.ShapeDtypeStruct((_NC * _N, _D), jnp.float32),
            jax.ShapeDtypeStruct((_NC * _N, 16), jnp.float32),
        ],
        mesh=plsc.VectorSubcoreMesh(core_axis_name="c", subcore_axis_name="s"),
        scratch_types=[
            pltpu.VMEM((_NCH, _C), jnp.int32),
            pltpu.VMEM((2, _C), jnp.int32),
            pltpu.VMEM((2, _C), jnp.int32),
            pltpu.VMEM((_C, _D), jnp.float32),
            pltpu.VMEM((_C, _D), jnp.float32),
            pltpu.VMEM((_C, 16), jnp.float32),
            pltpu.VMEM((1, _TPT), jnp.int32),
            pltpu.VMEM((_TPT, 16), jnp.float32),
            pltpu.VMEM_SHARED((_N, _D), jnp.float32),
            pltpu.VMEM_SHARED((_N, 16), jnp.float32),
            pltpu.SemaphoreType.DMA,
            pltpu.SemaphoreType.DMA,
        ],
        compiler_params=pltpu.CompilerParams(use_tc_tiling_on_sc=False),
    )
    feat_o, cnt_o = sc_call(pki, features, tid_p, twg, ones_c, zc)

    out = pl.pallas_call(
        _tc_body,
        out_shape=jax.ShapeDtypeStruct((_N, _OUT), jnp.float32),
    )(feat_o, cnt_o, W_proj, expert_protos, W_expert, W_reg)
    return out
